# baseline (speedup 1.0000x reference)
import jax
import jax.numpy as jnp
from jax import lax
from jax.experimental import pallas as pl
from jax.experimental.pallas import tpu as pltpu


def _pos_body(row_ref, col_ref, out_ref):
    col32 = col_ref[0:32, :]
    row32 = row_ref[0:32, :]
    left = jnp.broadcast_to(col32[None, :, :], (32, 32, 256)).reshape(1024, 256)
    right = jnp.broadcast_to(row32[:, None, :], (32, 32, 256)).reshape(1024, 256)
    out_ref[0, :, 0:256] = left
    out_ref[0, :, 256:512] = right
    for k in range(1, 4):
        out_ref[k, :, 0:256] = left
        out_ref[k, :, 256:512] = right


def kernel(x, row_embed, col_embed):
    b = x.shape[0]
    out = pl.pallas_call(
        _pos_body,
        grid=(b // 4,),
        in_specs=[
            pl.BlockSpec((50, 256), lambda i: (0, 0)),
            pl.BlockSpec((50, 256), lambda i: (0, 0)),
        ],
        out_specs=pl.BlockSpec((4, 1024, 512), lambda i: (i, 0, 0)),
        out_shape=jax.ShapeDtypeStruct((b, 1024, 512), jnp.float32),
    )(row_embed, col_embed)
    return jnp.transpose(out.reshape(b, 32, 32, 512), (0, 3, 1, 2))


# speedup vs baseline: 1.1839x; 1.1839x over previous
import jax
import jax.numpy as jnp
from jax import lax
from jax.experimental import pallas as pl
from jax.experimental.pallas import tpu as pltpu


def _pos_body(row_ref, col_ref, out_ref, pos_vmem, sem):
    col32 = col_ref[0:32, :]
    row32 = row_ref[0:32, :]
    left = jnp.broadcast_to(col32[None, :, :], (32, 32, 256)).reshape(1024, 256)
    right = jnp.broadcast_to(row32[:, None, :], (32, 32, 256)).reshape(1024, 256)
    pos_vmem[:, 0:256] = left
    pos_vmem[:, 256:512] = right
    copies = [
        pltpu.make_async_copy(pos_vmem, out_ref.at[b], sem)
        for b in range(out_ref.shape[0])
    ]
    for cp in copies:
        cp.start()
    for cp in copies:
        cp.wait()


def kernel(x, row_embed, col_embed):
    b = x.shape[0]
    out = pl.pallas_call(
        _pos_body,
        in_specs=[
            pl.BlockSpec(memory_space=pltpu.MemorySpace.VMEM),
            pl.BlockSpec(memory_space=pltpu.MemorySpace.VMEM),
        ],
        out_specs=pl.BlockSpec(memory_space=pl.ANY),
        out_shape=jax.ShapeDtypeStruct((b, 1024, 512), jnp.float32),
        scratch_shapes=[
            pltpu.VMEM((1024, 512), jnp.float32),
            pltpu.SemaphoreType.DMA,
        ],
    )(row_embed, col_embed)
    return jnp.transpose(out.reshape(b, 32, 32, 512), (0, 3, 1, 2))
